# baseline (speedup 1.0000x reference)
import functools

import jax
import jax.numpy as jnp
from jax import lax
from jax.experimental import pallas as pl
from jax.experimental.pallas import tpu as pltpu
from jax.experimental.pallas import tpu_sc as plsc

_W0 = 1000.0
_H0 = 1000.0

_NC = 2
_NS = 16
_NW = _NC * _NS
_CHUNK = 128


@functools.lru_cache(maxsize=None)
def _build(B, D, with_f):
    b_per_w = B // _NW
    assert B % _NW == 0 and b_per_w % _CHUNK == 0
    n_chunks = b_per_w // _CHUNK

    mesh = plsc.VectorSubcoreMesh(core_axis_name="c", subcore_axis_name="s")

    out_type = [jax.ShapeDtypeStruct((D, B), jnp.float32)]
    if with_f:
        out_type.append(jax.ShapeDtypeStruct((16,), jnp.float32))
    scratch = [
        pltpu.VMEM((n_chunks, _CHUNK), jnp.int32),
        pltpu.VMEM((D, b_per_w), jnp.float32),
        pltpu.SemaphoreType.DMA,
    ]
    if with_f:
        scratch.append(pltpu.VMEM((16,), jnp.float32))

    def body(*refs):
        if with_f:
            (p0, p1, p2, f_hbm, idx_hbm, tab_out, f_out,
             idx_v, rows_v, sem, f_v) = refs
        else:
            p0, p1, p2, idx_hbm, tab_out, idx_v, rows_v, sem = refs
        planes = (p0, p1, p2)
        wid = lax.axis_index("s") * _NC + lax.axis_index("c")
        base = wid * b_per_w

        pltpu.sync_copy(idx_hbm.at[pl.ds(wid * n_chunks, n_chunks)], idx_v)

        copies = []
        for c in range(D):
            for j in range(n_chunks):
                sl = pl.ds(j * _CHUNK, _CHUNK)
                copies.append(pltpu.async_copy(
                    planes[c].at[idx_v.at[j]], rows_v.at[c].at[sl], sem))
        for cp in copies:
            cp.wait()

        pltpu.sync_copy(rows_v, tab_out.at[:, pl.ds(base, b_per_w)])

        if with_f:
            @pl.when(wid == 0)
            def _():
                pltpu.sync_copy(f_hbm, f_v)
                fv = f_v[...]
                scale = jnp.where(lax.iota(jnp.int32, 16) == 0,
                                  jnp.float32(_W0), jnp.float32(_H0))
                f_v[...] = fv * fv * scale
                pltpu.sync_copy(f_v, f_out)

    return pl.kernel(
        body,
        mesh=mesh,
        out_type=out_type,
        scratch_types=scratch,
        compiler_params=pltpu.CompilerParams(use_tc_tiling_on_sc=False),
    )


def kernel(phi, t, f, indices):
    B = indices.shape[0]
    D = phi.shape[1]
    idx2 = indices.astype(jnp.int32).reshape(B // _CHUNK, _CHUNK)
    f16 = jnp.zeros((16,), jnp.float32).at[:2].set(f.astype(jnp.float32))
    phi_sel, fxy = _build(B, D, True)(
        phi[:, 0], phi[:, 1], phi[:, 2], f16, idx2)
    t_sel, = _build(B, D, False)(t[:, 0], t[:, 1], t[:, 2], idx2)
    return (phi_sel.T, t_sel.T, fxy[0], fxy[1])


# speedup vs baseline: 1.0322x; 1.0322x over previous
import functools

import jax
import jax.numpy as jnp
from jax import lax
from jax.experimental import pallas as pl
from jax.experimental.pallas import tpu as pltpu
from jax.experimental.pallas import tpu_sc as plsc

_W0 = 1000.0
_H0 = 1000.0

_NC = 2
_NS = 16
_NW = _NC * _NS
_CHUNK = 128


@functools.lru_cache(maxsize=None)
def _build(B, D, with_f):
    b_per_w = B // _NW
    assert B % _NW == 0 and b_per_w % _CHUNK == 0
    n_chunks = b_per_w // _CHUNK

    mesh = plsc.VectorSubcoreMesh(core_axis_name="c", subcore_axis_name="s")

    out_type = [jax.ShapeDtypeStruct((D, B), jnp.float32)]
    if with_f:
        out_type.append(jax.ShapeDtypeStruct((16,), jnp.float32))
    scratch = [
        pltpu.VMEM((n_chunks, _CHUNK), jnp.int32),
        pltpu.VMEM((D, b_per_w), jnp.float32),
        pltpu.SemaphoreType.DMA,
    ]
    if with_f:
        scratch.append(pltpu.VMEM((16,), jnp.float32))

    def body(*refs):
        if with_f:
            (tab_hbm, f_hbm, idx_hbm, tab_out, f_out,
             idx_v, rows_v, sem, f_v) = refs
        else:
            tab_hbm, idx_hbm, tab_out, idx_v, rows_v, sem = refs
        wid = lax.axis_index("s") * _NC + lax.axis_index("c")
        base = wid * b_per_w

        pltpu.sync_copy(idx_hbm.at[pl.ds(wid * n_chunks, n_chunks)], idx_v)

        copies = []
        for c in range(D):
            for j in range(n_chunks):
                sl = pl.ds(j * _CHUNK, _CHUNK)
                copies.append(pltpu.async_copy(
                    tab_hbm.at[c].at[idx_v.at[j]], rows_v.at[c].at[sl], sem))
        for cp in copies:
            cp.wait()

        pltpu.sync_copy(rows_v, tab_out.at[:, pl.ds(base, b_per_w)])

        if with_f:
            @pl.when(wid == 0)
            def _():
                pltpu.sync_copy(f_hbm, f_v)
                fv = f_v[...]
                scale = jnp.where(lax.iota(jnp.int32, 16) == 0,
                                  jnp.float32(_W0), jnp.float32(_H0))
                f_v[...] = fv * fv * scale
                pltpu.sync_copy(f_v, f_out)

    return pl.kernel(
        body,
        mesh=mesh,
        out_type=out_type,
        scratch_types=scratch,
        compiler_params=pltpu.CompilerParams(use_tc_tiling_on_sc=False),
    )


def kernel(phi, t, f, indices):
    B = indices.shape[0]
    D = phi.shape[1]
    idx2 = indices.astype(jnp.int32).reshape(B // _CHUNK, _CHUNK)
    f16 = jnp.zeros((16,), jnp.float32).at[:2].set(f.astype(jnp.float32))
    eye = jnp.eye(D, dtype=jnp.float32)
    phi_sel, fxy = _build(B, D, True)(
        jnp.einsum('ck,ik->ci', eye, phi), f16, idx2)
    t_sel, = _build(B, D, False)(jnp.einsum('ck,ik->ci', eye, t), idx2)
    return (phi_sel.T, t_sel.T, fxy[0], fxy[1])


# speedup vs baseline: 1.3164x; 1.2753x over previous
import functools

import jax
import jax.numpy as jnp
from jax import lax
from jax.experimental import pallas as pl
from jax.experimental.pallas import tpu as pltpu
from jax.experimental.pallas import tpu_sc as plsc

_W0 = 1000.0
_H0 = 1000.0

_NC = 2
_NS = 16
_NW = _NC * _NS
_CHUNK = 128


@functools.lru_cache(maxsize=None)
def _build(B, D, with_f):
    b_per_w = B // _NW
    assert B % _NW == 0 and b_per_w % _CHUNK == 0
    n_chunks = b_per_w // _CHUNK

    mesh = plsc.VectorSubcoreMesh(core_axis_name="c", subcore_axis_name="s")

    out_type = [jax.ShapeDtypeStruct((D, B), jnp.float32)]
    if with_f:
        out_type.append(jax.ShapeDtypeStruct((16,), jnp.float32))
    scratch = [
        pltpu.VMEM((n_chunks, _CHUNK), jnp.int32),
        pltpu.VMEM((D, b_per_w), jnp.float32),
        pltpu.SemaphoreType.DMA,
    ]
    if with_f:
        scratch.append(pltpu.VMEM((16,), jnp.float32))

    def body(*refs):
        if with_f:
            (tab_hbm, f_hbm, idx_hbm, tab_out, f_out,
             idx_v, rows_v, sem, f_v) = refs
        else:
            tab_hbm, idx_hbm, tab_out, idx_v, rows_v, sem = refs
        wid = lax.axis_index("s") * _NC + lax.axis_index("c")
        base = wid * b_per_w

        pltpu.sync_copy(idx_hbm.at[pl.ds(wid * n_chunks, n_chunks)], idx_v)

        copies = []
        for c in range(D):
            for j in range(n_chunks):
                sl = pl.ds(j * _CHUNK, _CHUNK)
                copies.append(pltpu.async_copy(
                    tab_hbm.at[c].at[idx_v.at[j]], rows_v.at[c].at[sl], sem))
        for cp in copies:
            cp.wait()

        pltpu.sync_copy(rows_v, tab_out.at[:, pl.ds(base, b_per_w)])

        if with_f:
            @pl.when(wid == 0)
            def _():
                pltpu.sync_copy(f_hbm, f_v)
                fv = f_v[...]
                scale = jnp.where(lax.iota(jnp.int32, 16) == 0,
                                  jnp.float32(_W0), jnp.float32(_H0))
                f_v[...] = fv * fv * scale
                pltpu.sync_copy(f_v, f_out)

    return pl.kernel(
        body,
        mesh=mesh,
        out_type=out_type,
        scratch_types=scratch,
        compiler_params=pltpu.CompilerParams(use_tc_tiling_on_sc=False),
    )


def kernel(phi, t, f, indices):
    B = indices.shape[0]
    D = phi.shape[1]
    idx2 = indices.astype(jnp.int32).reshape(B // _CHUNK, _CHUNK)
    f16 = jnp.zeros((16,), jnp.float32).at[:2].set(f.astype(jnp.float32))
    phi_sel, fxy = _build(B, D, True)(phi.T, f16, idx2)
    t_sel, = _build(B, D, False)(t.T, idx2)
    return (phi_sel.T, t_sel.T, fxy[0], fxy[1])
